# Initial kernel scaffold; baseline (speedup 1.0000x reference)
import jax
import jax.numpy as jnp
from jax.experimental import pallas as pl


def kernel(input_coords_cpu, input_resnames, input_atomnames, num_atoms):
    raise NotImplementedError("write your pallas kernel here")


# speedup vs baseline: 3.0894x; 3.0894x over previous
import functools

import jax
import jax.numpy as jnp
from jax import lax
from jax.experimental import pallas as pl
from jax.experimental.pallas import tpu as pltpu
from jax.experimental.pallas import tpu_sc as plsc

NUM_TYPES = 11
B = 1024
M = 4096
LANES = 16
NW = 32
SPW = B // NW
NVR = M // LANES

_mesh = plsc.VectorSubcoreMesh(core_axis_name="c", subcore_axis_name="s")


@functools.partial(
    pl.kernel,
    out_type=[
        jax.ShapeDtypeStruct((B, 3 * M), jnp.float32),
        jax.ShapeDtypeStruct((B * NUM_TYPES,), jnp.int32),
        jax.ShapeDtypeStruct((B * NUM_TYPES,), jnp.int32),
    ],
    mesh=_mesh,
    scratch_types=[
        pltpu.VMEM((M,), jnp.int32),
        pltpu.VMEM((M,), jnp.int32),
        pltpu.VMEM((3 * M,), jnp.float32),
        pltpu.VMEM((3 * M,), jnp.float32),
        pltpu.VMEM((M,), jnp.int32),
        pltpu.VMEM((M,), jnp.int32),
        pltpu.VMEM((LANES,), jnp.int32),
        pltpu.VMEM((LANES,), jnp.int32),
        pltpu.VMEM((SPW,), jnp.int32),
        pltpu.VMEM((SPW * NUM_TYPES,), jnp.int32),
        pltpu.VMEM((SPW * NUM_TYPES,), jnp.int32),
    ],
    compiler_params=pltpu.CompilerParams(needs_layout_passes=False),
)
def _typed_coords_sc(
    crd_hbm, res_hbm, atm_hbm, na_hbm,
    outc_hbm, cnts_hbm, offs_hbm,
    res_v, atm_v, crd_v, out_v, typ_v, rnk_v, cnt_v, off_v, na_v, co_v, of_v,
):
    wid = lax.axis_index("c") * 16 + lax.axis_index("s")
    base = wid * SPW
    pltpu.sync_copy(na_hbm.at[pl.ds(base, SPW)], na_v)
    iota = lax.broadcasted_iota(jnp.int32, (LANES,), 0)
    zero_f = jnp.zeros((LANES,), jnp.float32)

    def sample_body(si, _):
        row = base + si
        pltpu.sync_copy(res_hbm.at[row], res_v)
        pltpu.sync_copy(atm_hbm.at[row], atm_v)
        pltpu.sync_copy(crd_hbm.at[row], crd_v)
        cnt_v[...] = jnp.zeros((LANES,), jnp.int32)
        na_sp = plsc.load_gather(na_v, [jnp.zeros((LANES,), jnp.int32) + si])

        def pass_a(j, _):
            sl = pl.ds(j * LANES, LANES)
            t = (res_v[sl] + atm_v[sl]) % NUM_TYPES
            gi = j * LANES + iota
            t = jnp.where(gi < na_sp, t, NUM_TYPES)
            typ_v[sl] = t
            g = plsc.load_gather(cnt_v, [t])
            c1, lastm = plsc.scan_count(t)
            rnk_v[sl] = g + c1 - 1
            plsc.store_scatter(cnt_v, [t], g + c1, mask=lastm)
            return 0

        lax.fori_loop(0, NVR, pass_a, 0)

        c = cnt_v[...]
        off = plsc.cumsum(c) - c
        off_v[...] = off
        dsti = si * NUM_TYPES + iota
        m11 = iota < NUM_TYPES
        plsc.store_scatter(co_v, [dsti], c, mask=m11)
        plsc.store_scatter(of_v, [dsti], off, mask=m11)

        def pass_b(j, _):
            sl = pl.ds(j * LANES, LANES)
            t = typ_v[sl]
            pos = plsc.load_gather(off_v, [t]) + rnk_v[sl]
            src3 = (j * LANES) * 3 + iota * 3
            x = plsc.load_gather(crd_v, [src3])
            y = plsc.load_gather(crd_v, [src3 + 1])
            z = plsc.load_gather(crd_v, [src3 + 2])
            valid = t < NUM_TYPES
            x = jnp.where(valid, x, zero_f)
            y = jnp.where(valid, y, zero_f)
            z = jnp.where(valid, z, zero_f)
            d3 = pos * 3
            plsc.store_scatter(out_v, [d3], x)
            plsc.store_scatter(out_v, [d3 + 1], y)
            plsc.store_scatter(out_v, [d3 + 2], z)
            return 0

        lax.fori_loop(0, NVR, pass_b, 0)

        pltpu.sync_copy(out_v, outc_hbm.at[row])
        return 0

    lax.fori_loop(0, SPW, sample_body, 0)
    pltpu.sync_copy(co_v, cnts_hbm.at[pl.ds(base * NUM_TYPES, SPW * NUM_TYPES)])
    pltpu.sync_copy(of_v, offs_hbm.at[pl.ds(base * NUM_TYPES, SPW * NUM_TYPES)])


def kernel(input_coords_cpu, input_resnames, input_atomnames, num_atoms):
    out_coords, counts_flat, offsets_flat = _typed_coords_sc(
        input_coords_cpu,
        input_resnames.astype(jnp.int32),
        input_atomnames.astype(jnp.int32),
        num_atoms.astype(jnp.int32),
    )
    return (
        out_coords,
        counts_flat.reshape(B, NUM_TYPES),
        offsets_flat.reshape(B, NUM_TYPES),
    )


# speedup vs baseline: 3.6172x; 1.1709x over previous
import functools

import jax
import jax.numpy as jnp
from jax import lax
from jax.experimental import pallas as pl
from jax.experimental.pallas import tpu as pltpu
from jax.experimental.pallas import tpu_sc as plsc

NUM_TYPES = 11
B = 1024
M = 4096
LANES = 16
NW = 32
SPW = B // NW
NVR = M // LANES

_mesh = plsc.VectorSubcoreMesh(core_axis_name="c", subcore_axis_name="s")


@functools.partial(
    pl.kernel,
    out_type=[
        jax.ShapeDtypeStruct((B, 3 * M), jnp.float32),
        jax.ShapeDtypeStruct((B * NUM_TYPES,), jnp.int32),
        jax.ShapeDtypeStruct((B * NUM_TYPES,), jnp.int32),
    ],
    mesh=_mesh,
    scratch_types=[
        pltpu.VMEM((M,), jnp.int32),
        pltpu.VMEM((M,), jnp.int32),
        pltpu.VMEM((3 * M,), jnp.float32),
        pltpu.VMEM((3 * M,), jnp.float32),
        pltpu.VMEM((M,), jnp.int32),
        pltpu.VMEM((M,), jnp.int32),
        pltpu.VMEM((LANES,), jnp.int32),
        pltpu.VMEM((LANES,), jnp.int32),
        pltpu.VMEM((SPW,), jnp.int32),
        pltpu.VMEM((SPW * NUM_TYPES,), jnp.int32),
        pltpu.VMEM((SPW * NUM_TYPES,), jnp.int32),
    ],
    compiler_params=pltpu.CompilerParams(needs_layout_passes=False),
)
def _typed_coords_sc(
    crd_hbm, res_hbm, atm_hbm, na_hbm,
    outc_hbm, cnts_hbm, offs_hbm,
    res_v, atm_v, crd_v, out_v, typ_v, rnk_v, cnt_v, off_v, na_v, co_v, of_v,
):
    wid = lax.axis_index("c") * 16 + lax.axis_index("s")
    base = wid * SPW
    pltpu.sync_copy(na_hbm.at[pl.ds(base, SPW)], na_v)
    iota = lax.broadcasted_iota(jnp.int32, (LANES,), 0)
    zero_f = jnp.zeros((LANES,), jnp.float32)

    def sample_body(si, _):
        row = base + si
        pltpu.sync_copy(res_hbm.at[row], res_v)
        pltpu.sync_copy(atm_hbm.at[row], atm_v)
        pltpu.sync_copy(crd_hbm.at[row], crd_v)
        cnt_v[...] = jnp.zeros((LANES,), jnp.int32)
        na_sp = plsc.load_gather(na_v, [jnp.zeros((LANES,), jnp.int32) + si])

        def pass_a(j, _):
            sl = pl.ds(j * LANES, LANES)
            t = (res_v[sl] + atm_v[sl]) % NUM_TYPES
            gi = j * LANES + iota
            t = jnp.where(gi < na_sp, t, NUM_TYPES)
            typ_v[sl] = t
            g = plsc.load_gather(cnt_v, [t])
            c1, lastm = plsc.scan_count(t)
            rnk_v[sl] = g + c1 - 1
            plsc.store_scatter(cnt_v, [t], g + c1, mask=lastm)
            return 0

        lax.fori_loop(0, NVR, pass_a, 0, unroll=4)

        c = cnt_v[...]
        off = plsc.cumsum(c) - c
        off_v[...] = off
        dsti = si * NUM_TYPES + iota
        m11 = iota < NUM_TYPES
        plsc.store_scatter(co_v, [dsti], c, mask=m11)
        plsc.store_scatter(of_v, [dsti], off, mask=m11)

        @plsc.parallel_loop(0, NVR, unroll=4)
        def pass_b(j):
            sl = pl.ds(j * LANES, LANES)
            t = typ_v[sl]
            pos = plsc.load_gather(off_v, [t]) + rnk_v[sl]
            src3 = (j * LANES) * 3 + iota * 3
            x = plsc.load_gather(crd_v, [src3])
            y = plsc.load_gather(crd_v, [src3 + 1])
            z = plsc.load_gather(crd_v, [src3 + 2])
            valid = t < NUM_TYPES
            x = jnp.where(valid, x, zero_f)
            y = jnp.where(valid, y, zero_f)
            z = jnp.where(valid, z, zero_f)
            d3 = pos * 3
            plsc.store_scatter(out_v, [d3], x)
            plsc.store_scatter(out_v, [d3 + 1], y)
            plsc.store_scatter(out_v, [d3 + 2], z)

        pltpu.sync_copy(out_v, outc_hbm.at[row])
        return 0

    lax.fori_loop(0, SPW, sample_body, 0)
    pltpu.sync_copy(co_v, cnts_hbm.at[pl.ds(base * NUM_TYPES, SPW * NUM_TYPES)])
    pltpu.sync_copy(of_v, offs_hbm.at[pl.ds(base * NUM_TYPES, SPW * NUM_TYPES)])


def kernel(input_coords_cpu, input_resnames, input_atomnames, num_atoms):
    out_coords, counts_flat, offsets_flat = _typed_coords_sc(
        input_coords_cpu,
        input_resnames.astype(jnp.int32),
        input_atomnames.astype(jnp.int32),
        num_atoms.astype(jnp.int32),
    )
    return (
        out_coords,
        counts_flat.reshape(B, NUM_TYPES),
        offsets_flat.reshape(B, NUM_TYPES),
    )


# speedup vs baseline: 4.6633x; 1.2892x over previous
import functools

import jax
import jax.numpy as jnp
from jax import lax
from jax.experimental import pallas as pl
from jax.experimental.pallas import tpu as pltpu
from jax.experimental.pallas import tpu_sc as plsc

NUM_TYPES = 11
B = 1024
M = 4096
LANES = 16
NW = 32
SPW = B // NW
NVR = M // LANES

_mesh = plsc.VectorSubcoreMesh(core_axis_name="c", subcore_axis_name="s")


@functools.partial(
    pl.kernel,
    out_type=[
        jax.ShapeDtypeStruct((B, 3 * M), jnp.float32),
        jax.ShapeDtypeStruct((B * NUM_TYPES,), jnp.int32),
        jax.ShapeDtypeStruct((B * NUM_TYPES,), jnp.int32),
    ],
    mesh=_mesh,
    scratch_types=[
        pltpu.VMEM((M,), jnp.int32),
        pltpu.VMEM((M,), jnp.int32),
        pltpu.VMEM((3 * M,), jnp.float32),
        pltpu.VMEM((3 * M,), jnp.float32),
        pltpu.VMEM((M,), jnp.int32),
        pltpu.VMEM((M,), jnp.int32),
        pltpu.VMEM((M,), jnp.int32),
        pltpu.VMEM((LANES,), jnp.int32),
        pltpu.VMEM((SPW,), jnp.int32),
        pltpu.VMEM((SPW * NUM_TYPES,), jnp.int32),
        pltpu.VMEM((SPW * NUM_TYPES,), jnp.int32),
    ],
    compiler_params=pltpu.CompilerParams(needs_layout_passes=False),
)
def _typed_coords_sc(
    crd_hbm, res_hbm, atm_hbm, na_hbm,
    outc_hbm, cnts_hbm, offs_hbm,
    res_v, atm_v, crd_v, out_v, typ_v, rnk_v, hst_v, off_v, na_v, co_v, of_v,
):
    wid = lax.axis_index("c") * 16 + lax.axis_index("s")
    base = wid * SPW
    pltpu.sync_copy(na_hbm.at[pl.ds(base, SPW)], na_v)
    iota = lax.broadcasted_iota(jnp.int32, (LANES,), 0)
    zero_f = jnp.zeros((LANES,), jnp.float32)

    def sample_body(si, _):
        row = base + si
        pltpu.sync_copy(res_hbm.at[row], res_v)
        pltpu.sync_copy(atm_hbm.at[row], atm_v)
        pltpu.sync_copy(crd_hbm.at[row], crd_v)
        na_sp = plsc.load_gather(na_v, [jnp.zeros((LANES,), jnp.int32) + si])

        @plsc.parallel_loop(0, NVR, unroll=8)
        def zero_hist(j):
            hst_v[pl.ds(j * LANES, LANES)] = jnp.zeros((LANES,), jnp.int32)

        @plsc.parallel_loop(0, NVR, unroll=4)
        def pass_a(j):
            sl = pl.ds(j * LANES, LANES)
            t = (res_v[sl] + atm_v[sl]) % NUM_TYPES
            gi = j * LANES + iota
            t = jnp.where(gi < na_sp, t, NUM_TYPES)
            typ_v[sl] = t
            c1, lastm = plsc.scan_count(t)
            rnk_v[sl] = c1 - 1
            plsc.store_scatter(hst_v, [j * LANES + t], c1, mask=lastm)

        def prefix(j, acc):
            sl = pl.ds(j * LANES, LANES)
            hv = hst_v[sl]
            hst_v[sl] = acc
            return acc + hv

        c = lax.fori_loop(0, NVR, prefix, jnp.zeros((LANES,), jnp.int32),
                          unroll=4)
        off = plsc.cumsum(c) - c
        off_v[...] = off
        dsti = si * NUM_TYPES + iota
        m11 = iota < NUM_TYPES
        plsc.store_scatter(co_v, [dsti], c, mask=m11)
        plsc.store_scatter(of_v, [dsti], off, mask=m11)

        @plsc.parallel_loop(0, NVR, unroll=4)
        def pass_b(j):
            sl = pl.ds(j * LANES, LANES)
            t = typ_v[sl]
            pos = (plsc.load_gather(off_v, [t])
                   + plsc.load_gather(hst_v, [j * LANES + t])
                   + rnk_v[sl])
            src3 = (j * LANES) * 3 + iota * 3
            x = plsc.load_gather(crd_v, [src3])
            y = plsc.load_gather(crd_v, [src3 + 1])
            z = plsc.load_gather(crd_v, [src3 + 2])
            valid = t < NUM_TYPES
            x = jnp.where(valid, x, zero_f)
            y = jnp.where(valid, y, zero_f)
            z = jnp.where(valid, z, zero_f)
            d3 = pos * 3
            plsc.store_scatter(out_v, [d3], x)
            plsc.store_scatter(out_v, [d3 + 1], y)
            plsc.store_scatter(out_v, [d3 + 2], z)

        pltpu.sync_copy(out_v, outc_hbm.at[row])
        return 0

    lax.fori_loop(0, SPW, sample_body, 0)
    pltpu.sync_copy(co_v, cnts_hbm.at[pl.ds(base * NUM_TYPES, SPW * NUM_TYPES)])
    pltpu.sync_copy(of_v, offs_hbm.at[pl.ds(base * NUM_TYPES, SPW * NUM_TYPES)])


def kernel(input_coords_cpu, input_resnames, input_atomnames, num_atoms):
    out_coords, counts_flat, offsets_flat = _typed_coords_sc(
        input_coords_cpu,
        input_resnames.astype(jnp.int32),
        input_atomnames.astype(jnp.int32),
        num_atoms.astype(jnp.int32),
    )
    return (
        out_coords,
        counts_flat.reshape(B, NUM_TYPES),
        offsets_flat.reshape(B, NUM_TYPES),
    )


# speedup vs baseline: 5.7911x; 1.2418x over previous
import functools

import jax
import jax.numpy as jnp
from jax import lax
from jax.experimental import pallas as pl
from jax.experimental.pallas import tpu as pltpu
from jax.experimental.pallas import tpu_sc as plsc

NUM_TYPES = 11
B = 1024
M = 4096
LANES = 16
NW = 32
SPW = B // NW
NVR = M // LANES

_mesh = plsc.VectorSubcoreMesh(core_axis_name="c", subcore_axis_name="s")


@functools.partial(
    pl.kernel,
    out_type=[
        jax.ShapeDtypeStruct((B, 3 * M), jnp.float32),
        jax.ShapeDtypeStruct((B * NUM_TYPES,), jnp.int32),
        jax.ShapeDtypeStruct((B * NUM_TYPES,), jnp.int32),
    ],
    mesh=_mesh,
    scratch_types=[
        pltpu.VMEM((M,), jnp.int32),
        pltpu.VMEM((M,), jnp.int32),
        pltpu.VMEM((M,), jnp.int32),
        pltpu.VMEM((M,), jnp.int32),
        pltpu.VMEM((3 * M,), jnp.float32),
        pltpu.VMEM((3 * M,), jnp.float32),
        pltpu.VMEM((3 * M,), jnp.float32),
        pltpu.VMEM((3 * M,), jnp.float32),
        pltpu.VMEM((M,), jnp.int32),
        pltpu.VMEM((M,), jnp.int32),
        pltpu.VMEM((M,), jnp.int32),
        pltpu.VMEM((LANES,), jnp.int32),
        pltpu.VMEM((SPW,), jnp.int32),
        pltpu.VMEM((SPW * NUM_TYPES,), jnp.int32),
        pltpu.VMEM((SPW * NUM_TYPES,), jnp.int32),
        pltpu.SemaphoreType.DMA,
        pltpu.SemaphoreType.DMA,
        pltpu.SemaphoreType.DMA,
        pltpu.SemaphoreType.DMA,
    ],
    compiler_params=pltpu.CompilerParams(needs_layout_passes=False),
)
def _typed_coords_sc(
    crd_hbm, res_hbm, atm_hbm, na_hbm,
    outc_hbm, cnts_hbm, offs_hbm,
    res_v0, res_v1, atm_v0, atm_v1, crd_v0, crd_v1, out_v0, out_v1,
    typ_v, rnk_v, hst_v, off_v, na_v, co_v, of_v,
    sin0, sin1, sout0, sout1,
):
    res_b = (res_v0, res_v1)
    atm_b = (atm_v0, atm_v1)
    crd_b = (crd_v0, crd_v1)
    out_b = (out_v0, out_v1)
    sin = (sin0, sin1)
    sout = (sout0, sout1)

    wid = lax.axis_index("c") * 16 + lax.axis_index("s")
    base = wid * SPW
    pltpu.sync_copy(na_hbm.at[pl.ds(base, SPW)], na_v)
    iota = lax.broadcasted_iota(jnp.int32, (LANES,), 0)
    zero_f = jnp.zeros((LANES,), jnp.float32)

    def issue_in(si, k):
        row = base + si
        pltpu.async_copy(res_hbm.at[row], res_b[k], sin[k])
        pltpu.async_copy(atm_hbm.at[row], atm_b[k], sin[k])
        pltpu.async_copy(crd_hbm.at[row], crd_b[k], sin[k])

    def wait_in(si, k):
        row = base + si
        pltpu.make_async_copy(res_hbm.at[row], res_b[k], sin[k]).wait()
        pltpu.make_async_copy(atm_hbm.at[row], atm_b[k], sin[k]).wait()
        pltpu.make_async_copy(crd_hbm.at[row], crd_b[k], sin[k]).wait()

    def compute_sample(si, k):
        res_v, atm_v, crd_v, out_v = res_b[k], atm_b[k], crd_b[k], out_b[k]
        na_sp = plsc.load_gather(na_v, [jnp.zeros((LANES,), jnp.int32) + si])

        @plsc.parallel_loop(0, NVR, unroll=8)
        def zero_hist(j):
            hst_v[pl.ds(j * LANES, LANES)] = jnp.zeros((LANES,), jnp.int32)

        @plsc.parallel_loop(0, NVR, unroll=4)
        def pass_a(j):
            sl = pl.ds(j * LANES, LANES)
            t = (res_v[sl] + atm_v[sl]) % NUM_TYPES
            gi = j * LANES + iota
            t = jnp.where(gi < na_sp, t, NUM_TYPES)
            typ_v[sl] = t
            c1, lastm = plsc.scan_count(t)
            rnk_v[sl] = c1 - 1
            plsc.store_scatter(hst_v, [j * LANES + t], c1, mask=lastm)

        def prefix(j, acc):
            sl = pl.ds(j * LANES, LANES)
            hv = hst_v[sl]
            hst_v[sl] = acc
            return acc + hv

        c = lax.fori_loop(0, NVR, prefix, jnp.zeros((LANES,), jnp.int32),
                          unroll=4)
        off = plsc.cumsum(c) - c
        off_v[...] = off
        dsti = si * NUM_TYPES + iota
        m11 = iota < NUM_TYPES
        plsc.store_scatter(co_v, [dsti], c, mask=m11)
        plsc.store_scatter(of_v, [dsti], off, mask=m11)

        @pl.when(si >= 2)
        def _():
            pltpu.make_async_copy(out_v, outc_hbm.at[base + si - 2],
                                  sout[k]).wait()

        @plsc.parallel_loop(0, NVR, unroll=4)
        def pass_b(j):
            sl = pl.ds(j * LANES, LANES)
            t = typ_v[sl]
            pos = (plsc.load_gather(off_v, [t])
                   + plsc.load_gather(hst_v, [j * LANES + t])
                   + rnk_v[sl])
            src3 = (j * LANES) * 3 + iota * 3
            x = plsc.load_gather(crd_v, [src3])
            y = plsc.load_gather(crd_v, [src3 + 1])
            z = plsc.load_gather(crd_v, [src3 + 2])
            valid = t < NUM_TYPES
            x = jnp.where(valid, x, zero_f)
            y = jnp.where(valid, y, zero_f)
            z = jnp.where(valid, z, zero_f)
            d3 = pos * 3
            plsc.store_scatter(out_v, [d3], x)
            plsc.store_scatter(out_v, [d3 + 1], y)
            plsc.store_scatter(out_v, [d3 + 2], z)

        pltpu.async_copy(out_v, outc_hbm.at[base + si], sout[k])

    issue_in(0, 0)

    def pair_body(p, _):
        for k in (0, 1):
            si = p * 2 + k

            @pl.when(si + 1 < SPW)
            def _():
                issue_in(si + 1, 1 - k)

            wait_in(si, k)
            compute_sample(si, k)
        return 0

    lax.fori_loop(0, SPW // 2, pair_body, 0)
    pltpu.make_async_copy(out_b[0], outc_hbm.at[base + SPW - 2], sout[0]).wait()
    pltpu.make_async_copy(out_b[1], outc_hbm.at[base + SPW - 1], sout[1]).wait()
    pltpu.sync_copy(co_v, cnts_hbm.at[pl.ds(base * NUM_TYPES, SPW * NUM_TYPES)])
    pltpu.sync_copy(of_v, offs_hbm.at[pl.ds(base * NUM_TYPES, SPW * NUM_TYPES)])


def kernel(input_coords_cpu, input_resnames, input_atomnames, num_atoms):
    out_coords, counts_flat, offsets_flat = _typed_coords_sc(
        input_coords_cpu,
        input_resnames.astype(jnp.int32),
        input_atomnames.astype(jnp.int32),
        num_atoms.astype(jnp.int32),
    )
    return (
        out_coords,
        counts_flat.reshape(B, NUM_TYPES),
        offsets_flat.reshape(B, NUM_TYPES),
    )
